# baseline (speedup 1.0000x reference)
import functools

import jax
import jax.numpy as jnp
from jax import lax
from jax.experimental import pallas as pl
from jax.experimental.pallas import tpu as pltpu
from jax.experimental.pallas import tpu_sc as plsc

_TOKENS = 16384
_D = 512
_B = 16
_BLK = 4096
_NBLK = _TOKENS // _BLK

_NC = 2
_NS = 16
_L = 16
_NW = _NC * _NS
_RPW = _TOKENS // _NW
_CHUNK = 64
_NCHUNK = _RPW // _CHUNK


_NJ = _D // _L


def _sc_body(splits_hbm, x_hbm, out_hbm, splits_v, chunk_v, sums_v,
             sem0, sem1):
    c = lax.axis_index("c")
    s = lax.axis_index("s")
    w = c * _NS + s
    base = w * _RPW

    pltpu.sync_copy(splits_hbm, splits_v)

    for b in range(_B):
        for k in range(_NJ):
            sums_v[b, pl.ds(k * _L, _L)] = jnp.zeros((_L,), jnp.float32)

    bounds = [splits_v[j, :][0] for j in range(_B + 1)]

    sems = (sem0, sem1)

    def _copy(k):
        return pltpu.make_async_copy(
            x_hbm.at[pl.ds(base + k * _CHUNK, _CHUNK)],
            chunk_v.at[k % 2],
            sems[k % 2],
        )

    zero = jnp.int32(0)
    one = jnp.int32(1)

    def _seg_span(row0, row_last):
        slo = zero
        shi = zero
        for j in range(1, _B + 1):
            slo = slo + jnp.where(bounds[j] <= row0, one, zero)
            shi = shi + jnp.where(bounds[j] <= row_last, one, zero)
        return slo, shi

    _copy(0).start()
    for k in range(_NCHUNK):
        if k + 1 < _NCHUNK:
            _copy(k + 1).start()
        _copy(k).wait()
        row0 = base + k * _CHUNK
        slo, shi = _seg_span(row0, row0 + _CHUNK - 1)

        def _seg(sg, carry, k=k):
            lo = jnp.clip(splits_v[sg, :][0] - row0, 0, _CHUNK)
            hi = jnp.clip(splits_v[sg + 1, :][0] - row0, 0, _CHUNK)

            def _row(r, accs):
                return tuple(
                    accs[j] + chunk_v[k % 2, r, pl.ds(j * _L, _L)]
                    for j in range(_NJ)
                )

            zeros = tuple(jnp.zeros((_L,), jnp.float32) for _ in range(_NJ))
            accs = lax.fori_loop(lo, hi, _row, zeros)
            for j in range(_NJ):
                sums_v[sg, pl.ds(j * _L, _L)] = (
                    sums_v[sg, pl.ds(j * _L, _L)] + accs[j]
                )
            return carry

        lax.fori_loop(slo, shi + 1, _seg, zero)

    pltpu.sync_copy(sums_v, out_hbm.at[pl.ds(w * _B, _B)])


def _sc_partial_sums(splits_rep, x_data):
    mesh = plsc.VectorSubcoreMesh(core_axis_name="c", subcore_axis_name="s")
    run = pl.kernel(
        _sc_body,
        out_type=jax.ShapeDtypeStruct((_NW * _B, _D), jnp.float32),
        mesh=mesh,
        scratch_types=[
            pltpu.VMEM((_B + 1, _L), jnp.int32),
            pltpu.VMEM((2, _CHUNK, _D), jnp.float32),
            pltpu.VMEM((_B, _D), jnp.float32),
            pltpu.SemaphoreType.DMA,
            pltpu.SemaphoreType.DMA,
        ],
    )
    return run(splits_rep, x_data)


def _onehot(splits_row, j, blk, nseg):
    rows = lax.broadcasted_iota(jnp.int32, (blk, nseg), 0) + j * blk
    lower = jnp.broadcast_to(splits_row[:, :nseg], (blk, nseg))
    upper = jnp.broadcast_to(splits_row[:, 1:], (blk, nseg))
    return ((rows >= lower) & (rows < upper)).astype(jnp.float32)


def _tc_body(splits_ref, part_ref, x_ref, out_ref, means_ref):
    j = pl.program_id(0)
    splits_row = splits_ref[:]

    @pl.when(j == 0)
    def _finalize():
        counts = (splits_row[0, 1:] - splits_row[0, :_B]).astype(jnp.float32)
        denom = jnp.maximum(counts, 1.0)[:, None]
        sums = part_ref[pl.ds(0, _B), :]
        for w in range(1, _NW):
            sums = sums + part_ref[pl.ds(w * _B, _B), :]
        means_ref[:] = sums / denom

    oneh = _onehot(splits_row, j, _BLK, _B)
    out_ref[:, :_D] = lax.dot_general(
        oneh, means_ref[:],
        dimension_numbers=(((1,), (0,)), ((), ())),
        preferred_element_type=jnp.float32,
    )
    out_ref[:, _D:] = x_ref[:]


def _tc_broadcast_concat(splits, partial, x_data):
    return pl.pallas_call(
        _tc_body,
        grid=(_NBLK,),
        in_specs=[
            pl.BlockSpec((1, _B + 1), lambda j: (0, 0)),
            pl.BlockSpec((_NW * _B, _D), lambda j: (0, 0)),
            pl.BlockSpec((_BLK, _D), lambda j: (j, 0)),
        ],
        out_specs=pl.BlockSpec((_BLK, 2 * _D), lambda j: (j, 0)),
        out_shape=jax.ShapeDtypeStruct((_TOKENS, 2 * _D), jnp.float32),
        scratch_shapes=[pltpu.VMEM((_B, _D), jnp.float32)],
    )(splits, partial, x_data)


def kernel(x_data, x_row_splits):
    splits32 = x_row_splits.astype(jnp.int32)
    splits = splits32.reshape(1, _B + 1)
    splits_rep = jnp.broadcast_to(splits32[:, None], (_B + 1, _L))
    partial = _sc_partial_sums(splits_rep, x_data)
    return _tc_broadcast_concat(splits, partial, x_data)


# speedup vs baseline: 1.1037x; 1.1037x over previous
import functools

import jax
import jax.numpy as jnp
from jax import lax
from jax.experimental import pallas as pl
from jax.experimental.pallas import tpu as pltpu
from jax.experimental.pallas import tpu_sc as plsc

_TOKENS = 16384
_D = 512
_B = 16
_BLK = 4096
_NBLK = _TOKENS // _BLK

_NC = 2
_NS = 16
_L = 16
_NW = _NC * _NS
_RPW = _TOKENS // _NW
_CHUNK = 64
_NCHUNK = _RPW // _CHUNK
_NBUF = 3


_NJ = _D // _L


def _sc_body(splits_hbm, x_hbm, out_hbm, splits_v, chunk_v, sums_v,
             sem0, sem1, sem2):
    c = lax.axis_index("c")
    s = lax.axis_index("s")
    w = c * _NS + s
    base = w * _RPW

    pltpu.sync_copy(splits_hbm, splits_v)

    for b in range(_B):
        for k in range(_NJ):
            sums_v[b, pl.ds(k * _L, _L)] = jnp.zeros((_L,), jnp.float32)

    bounds = [splits_v[j, :][0] for j in range(_B + 1)]

    sems = (sem0, sem1, sem2)

    def _copy(k):
        return pltpu.make_async_copy(
            x_hbm.at[pl.ds(base + k * _CHUNK, _CHUNK)],
            chunk_v.at[k % _NBUF],
            sems[k % _NBUF],
        )

    zero = jnp.int32(0)
    one = jnp.int32(1)

    def _seg_span(row0, row_last):
        slo = zero
        shi = zero
        for j in range(1, _B + 1):
            slo = slo + jnp.where(bounds[j] <= row0, one, zero)
            shi = shi + jnp.where(bounds[j] <= row_last, one, zero)
        return slo, shi

    for k in range(_NBUF - 1):
        _copy(k).start()
    for k in range(_NCHUNK):
        if k + _NBUF - 1 < _NCHUNK:
            _copy(k + _NBUF - 1).start()
        _copy(k).wait()
        row0 = base + k * _CHUNK
        slo, shi = _seg_span(row0, row0 + _CHUNK - 1)

        def _seg(sg, carry, k=k):
            lo = jnp.clip(splits_v[sg, :][0] - row0, 0, _CHUNK)
            hi = jnp.clip(splits_v[sg + 1, :][0] - row0, 0, _CHUNK)

            def _row(r, accs):
                return tuple(
                    accs[j] + chunk_v[k % _NBUF, r, pl.ds(j * _L, _L)]
                    for j in range(_NJ)
                )

            zeros = tuple(jnp.zeros((_L,), jnp.float32) for _ in range(_NJ))
            accs = lax.fori_loop(lo, hi, _row, zeros)
            for j in range(_NJ):
                sums_v[sg, pl.ds(j * _L, _L)] = (
                    sums_v[sg, pl.ds(j * _L, _L)] + accs[j]
                )
            return carry

        lax.fori_loop(slo, shi + 1, _seg, zero)

    pltpu.sync_copy(sums_v, out_hbm.at[pl.ds(w * _B, _B)])


def _sc_partial_sums(splits_rep, x_data):
    mesh = plsc.VectorSubcoreMesh(core_axis_name="c", subcore_axis_name="s")
    run = pl.kernel(
        _sc_body,
        out_type=jax.ShapeDtypeStruct((_NW * _B, _D), jnp.float32),
        mesh=mesh,
        scratch_types=[
            pltpu.VMEM((_B + 1, _L), jnp.int32),
            pltpu.VMEM((_NBUF, _CHUNK, _D), jnp.float32),
            pltpu.VMEM((_B, _D), jnp.float32),
            pltpu.SemaphoreType.DMA,
            pltpu.SemaphoreType.DMA,
            pltpu.SemaphoreType.DMA,
        ],
    )
    return run(splits_rep, x_data)


def _onehot(splits_row, j, blk, nseg):
    rows = lax.broadcasted_iota(jnp.int32, (blk, nseg), 0) + j * blk
    lower = jnp.broadcast_to(splits_row[:, :nseg], (blk, nseg))
    upper = jnp.broadcast_to(splits_row[:, 1:], (blk, nseg))
    return ((rows >= lower) & (rows < upper)).astype(jnp.float32)


def _copy_body(x_ref, out_ref):
    out_ref[:] = x_ref[:]


def _tc_copy_half(x_data):
    return pl.pallas_call(
        _copy_body,
        grid=(_NBLK,),
        in_specs=[pl.BlockSpec((_BLK, _D), lambda j: (j, 0))],
        out_specs=pl.BlockSpec((_BLK, _D), lambda j: (j, 1)),
        out_shape=jax.ShapeDtypeStruct((_TOKENS, 2 * _D), jnp.float32),
    )(x_data)


def _means_body(splits_ref, part_ref, prev_ref, out_ref, means_ref):
    del prev_ref
    j = pl.program_id(0)
    splits_row = splits_ref[:]

    @pl.when(j == 0)
    def _finalize():
        counts = (splits_row[0, 1:] - splits_row[0, :_B]).astype(jnp.float32)
        denom = jnp.maximum(counts, 1.0)[:, None]
        sums = part_ref[pl.ds(0, _B), :]
        for w in range(1, _NW):
            sums = sums + part_ref[pl.ds(w * _B, _B), :]
        means_ref[:] = sums / denom

    oneh = _onehot(splits_row, j, _BLK, _B)
    out_ref[:] = lax.dot_general(
        oneh, means_ref[:],
        dimension_numbers=(((1,), (0,)), ((), ())),
        preferred_element_type=jnp.float32,
    )


def _tc_means_half(splits, partial, prev_out):
    return pl.pallas_call(
        _means_body,
        grid=(_NBLK,),
        in_specs=[
            pl.BlockSpec((1, _B + 1), lambda j: (0, 0)),
            pl.BlockSpec((_NW * _B, _D), lambda j: (0, 0)),
            pl.BlockSpec(memory_space=pl.ANY),
        ],
        out_specs=pl.BlockSpec((_BLK, _D), lambda j: (j, 0)),
        out_shape=jax.ShapeDtypeStruct((_TOKENS, 2 * _D), jnp.float32),
        scratch_shapes=[pltpu.VMEM((_B, _D), jnp.float32)],
        input_output_aliases={2: 0},
    )(splits, partial, prev_out)


def kernel(x_data, x_row_splits):
    splits32 = x_row_splits.astype(jnp.int32)
    splits = splits32.reshape(1, _B + 1)
    splits_rep = jnp.broadcast_to(splits32[:, None], (_B + 1, _L))
    partial = _sc_partial_sums(splits_rep, x_data)
    out = _tc_copy_half(x_data)
    return _tc_means_half(splits, partial, out)
